# baseline (speedup 1.0000x reference)
import functools

import jax
import jax.numpy as jnp
from jax import lax
from jax.experimental import pallas as pl
from jax.experimental.pallas import tpu as pltpu
from jax.experimental.pallas import tpu_sc as plsc

NUM_EMBED = 100000
EMBED_DIM = 128
COL_DIM = 32
BATCH = 16384

_info = plsc.get_sparse_core_info()
_NC, _NS, _L = _info.num_cores, _info.num_subcores, _info.num_lanes
_NW = _NC * _NS
_PAIR = BATCH // _NS
_R0 = 544
_R1 = _PAIR - _R0
_NCH = 4
_UNROLL = 8
_AHEAD = 3


def _make_kernel():
    mesh = plsc.VectorSubcoreMesh(core_axis_name="c", subcore_axis_name="s")

    @functools.partial(
        pl.kernel,
        mesh=mesh,
        out_type=jax.ShapeDtypeStruct((BATCH, EMBED_DIM), jnp.float32),
        scratch_types=[
            pltpu.VMEM((_R0,), jnp.int32),
            pltpu.VMEM((_R0, EMBED_DIM), jnp.float32),
            pltpu.VMEM((COL_DIM,), jnp.float32),
        ]
        + [pltpu.SemaphoreType.DMA] * (_NCH + 2),
    )
    def k(x_hbm, table_hbm, se_hbm, out_hbm, idx_v, rows_v, se_v, *sems):
        gsems, wsem, ssem = sems[:_NCH], sems[_NCH], sems[_NCH + 1]
        cid = lax.axis_index("c")
        sid = lax.axis_index("s")
        se_copy = pltpu.async_copy(se_hbm, se_v, ssem)

        def pipeline(base, size):
            ch = size // _NCH
            pltpu.sync_copy(x_hbm.at[pl.ds(base, ch)], idx_v.at[pl.ds(0, ch)])

            def issue_gather(c):
                return pltpu.async_copy(
                    table_hbm.at[idx_v.at[pl.ds(c * ch, ch)]],
                    rows_v.at[pl.ds(c * ch, ch)],
                    gsems[c],
                )

            gathers = [issue_gather(0)]
            pltpu.sync_copy(
                x_hbm.at[pl.ds(base + ch, size - ch)],
                idx_v.at[pl.ds(ch, size - ch)],
            )
            gathers += [issue_gather(c) for c in range(1, _AHEAD)]
            se_copy.wait()
            se_lo = se_v[pl.ds(0, _L)]
            se_hi = se_v[pl.ds(_L, _L)]

            writes = []
            for c in range(_NCH):
                gathers[c].wait()

                def overwrite(i, carry, c=c):
                    r = c * ch + i * _UNROLL
                    for j in range(_UNROLL):
                        rows_v[r + j, pl.ds(0, _L)] = se_lo
                        rows_v[r + j, pl.ds(_L, _L)] = se_hi
                    return carry

                lax.fori_loop(0, ch // _UNROLL, overwrite, 0)
                writes.append(
                    pltpu.async_copy(
                        rows_v.at[pl.ds(c * ch, ch)],
                        out_hbm.at[pl.ds(base + c * ch, ch)],
                        wsem,
                    )
                )
                if c + _AHEAD < _NCH:
                    gathers.append(issue_gather(c + _AHEAD))
            for w in writes:
                w.wait()

        @pl.when(cid == 0)
        def _():
            pipeline(sid * _PAIR, _R0)

        @pl.when(cid == 1)
        def _():
            pipeline(sid * _PAIR + _R0, _R1)

    return k


_sc_kernel = _make_kernel()


def kernel(X, embed_weight, shared_embed):
    idx = X.astype(jnp.int32)
    se = shared_embed.reshape((COL_DIM,))
    return _sc_kernel(idx, embed_weight, se)


# speedup vs baseline: 1.0044x; 1.0044x over previous
import functools

import jax
import jax.numpy as jnp
from jax import lax
from jax.experimental import pallas as pl
from jax.experimental.pallas import tpu as pltpu
from jax.experimental.pallas import tpu_sc as plsc

NUM_EMBED = 100000
EMBED_DIM = 128
COL_DIM = 32
BATCH = 16384

_info = plsc.get_sparse_core_info()
_NC, _NS, _L = _info.num_cores, _info.num_subcores, _info.num_lanes
_NW = _NC * _NS
_PAIR = BATCH // _NS
_R0 = 480
_R1 = _PAIR - _R0
_RMAX = max(_R0, _R1)
_NCH = 4
_UNROLL = 8
_AHEAD = 3


def _make_kernel():
    mesh = plsc.VectorSubcoreMesh(core_axis_name="c", subcore_axis_name="s")

    @functools.partial(
        pl.kernel,
        mesh=mesh,
        out_type=jax.ShapeDtypeStruct((BATCH, EMBED_DIM), jnp.float32),
        scratch_types=[
            pltpu.VMEM((_RMAX,), jnp.int32),
            pltpu.VMEM((_RMAX, EMBED_DIM), jnp.float32),
            pltpu.VMEM((COL_DIM,), jnp.float32),
        ]
        + [pltpu.SemaphoreType.DMA] * (_NCH + 2),
    )
    def k(x_hbm, table_hbm, se_hbm, out_hbm, idx_v, rows_v, se_v, *sems):
        gsems, wsem, ssem = sems[:_NCH], sems[_NCH], sems[_NCH + 1]
        cid = lax.axis_index("c")
        sid = lax.axis_index("s")
        se_copy = pltpu.async_copy(se_hbm, se_v, ssem)

        def pipeline(base, size):
            ch = size // _NCH
            pltpu.sync_copy(x_hbm.at[pl.ds(base, ch)], idx_v.at[pl.ds(0, ch)])

            def issue_gather(c):
                return pltpu.async_copy(
                    table_hbm.at[idx_v.at[pl.ds(c * ch, ch)]],
                    rows_v.at[pl.ds(c * ch, ch)],
                    gsems[c],
                )

            gathers = [issue_gather(0)]
            pltpu.sync_copy(
                x_hbm.at[pl.ds(base + ch, size - ch)],
                idx_v.at[pl.ds(ch, size - ch)],
            )
            gathers += [issue_gather(c) for c in range(1, _AHEAD)]
            se_copy.wait()
            se_lo = se_v[pl.ds(0, _L)]
            se_hi = se_v[pl.ds(_L, _L)]

            writes = []
            for c in range(_NCH):
                gathers[c].wait()

                def overwrite(i, carry, c=c):
                    r = c * ch + i * _UNROLL
                    for j in range(_UNROLL):
                        rows_v[r + j, pl.ds(0, _L)] = se_lo
                        rows_v[r + j, pl.ds(_L, _L)] = se_hi
                    return carry

                lax.fori_loop(0, ch // _UNROLL, overwrite, 0)
                writes.append(
                    pltpu.async_copy(
                        rows_v.at[pl.ds(c * ch, ch)],
                        out_hbm.at[pl.ds(base + c * ch, ch)],
                        wsem,
                    )
                )
                if c + _AHEAD < _NCH:
                    gathers.append(issue_gather(c + _AHEAD))
            for w in writes:
                w.wait()

        @pl.when(cid == 0)
        def _():
            pipeline(sid * _PAIR, _R0)

        @pl.when(cid == 1)
        def _():
            pipeline(sid * _PAIR + _R0, _R1)

    return k


_sc_kernel = _make_kernel()


def kernel(X, embed_weight, shared_embed):
    idx = X.astype(jnp.int32)
    se = shared_embed.reshape((COL_DIM,))
    return _sc_kernel(idx, embed_weight, se)


# speedup vs baseline: 1.0264x; 1.0220x over previous
import functools

import jax
import jax.numpy as jnp
from jax import lax
from jax.experimental import pallas as pl
from jax.experimental.pallas import tpu as pltpu
from jax.experimental.pallas import tpu_sc as plsc

NUM_EMBED = 100000
EMBED_DIM = 128
COL_DIM = 32
BATCH = 16384

_info = plsc.get_sparse_core_info()
_NC, _NS, _L = _info.num_cores, _info.num_subcores, _info.num_lanes
_NW = _NC * _NS
_B_PER_W = BATCH // _NW
_NCH = 4
_CH = _B_PER_W // _NCH
_UNROLL = 8
_AHEAD = 3


def _make_kernel():
    mesh = plsc.VectorSubcoreMesh(core_axis_name="c", subcore_axis_name="s")

    @functools.partial(
        pl.kernel,
        mesh=mesh,
        out_type=jax.ShapeDtypeStruct((BATCH, EMBED_DIM), jnp.float32),
        scratch_types=[
            pltpu.VMEM((_B_PER_W,), jnp.int32),
            pltpu.VMEM((_B_PER_W, EMBED_DIM), jnp.float32),
            pltpu.VMEM((COL_DIM,), jnp.float32),
        ]
        + [pltpu.SemaphoreType.DMA] * (_NCH + 2),
    )
    def k(x_hbm, table_hbm, se_hbm, out_hbm, idx_v, rows_v, se_v, *sems):
        gsems, wsem, ssem = sems[:_NCH], sems[_NCH], sems[_NCH + 1]
        wid = lax.axis_index("s") * _NC + lax.axis_index("c")
        base = wid * _B_PER_W
        se_copy = pltpu.async_copy(se_hbm, se_v, ssem)
        pltpu.sync_copy(x_hbm.at[pl.ds(base, _CH)], idx_v.at[pl.ds(0, _CH)])

        def issue_gather(c):
            return pltpu.async_copy(
                table_hbm.at[idx_v.at[pl.ds(c * _CH, _CH)]],
                rows_v.at[pl.ds(c * _CH, _CH)],
                gsems[c],
            )

        gathers = [issue_gather(0)]
        pltpu.sync_copy(
            x_hbm.at[pl.ds(base + _CH, _B_PER_W - _CH)],
            idx_v.at[pl.ds(_CH, _B_PER_W - _CH)],
        )
        gathers += [issue_gather(c) for c in range(1, _AHEAD)]
        se_copy.wait()
        se_lo = se_v[pl.ds(0, _L)]
        se_hi = se_v[pl.ds(_L, _L)]

        writes = []
        for c in range(_NCH):
            gathers[c].wait()

            def overwrite(i, carry, c=c):
                r = c * _CH + i * _UNROLL
                for j in range(_UNROLL):
                    rows_v[r + j, pl.ds(0, _L)] = se_lo
                    rows_v[r + j, pl.ds(_L, _L)] = se_hi
                return carry

            lax.fori_loop(0, _CH // _UNROLL, overwrite, 0)
            writes.append(
                pltpu.async_copy(
                    rows_v.at[pl.ds(c * _CH, _CH)],
                    out_hbm.at[pl.ds(base + c * _CH, _CH)],
                    wsem,
                )
            )
            if c + _AHEAD < _NCH:
                gathers.append(issue_gather(c + _AHEAD))
        for w in writes:
            w.wait()

    return k


_sc_kernel = _make_kernel()


def kernel(X, embed_weight, shared_embed):
    idx = X.astype(jnp.int32)
    se = shared_embed.reshape((COL_DIM,))
    return _sc_kernel(idx, embed_weight, se)
